# baseline (speedup 1.0000x reference)
import functools

import jax
import jax.numpy as jnp
from jax import lax
from jax.experimental import pallas as pl
from jax.experimental.pallas import tpu as pltpu
from jax.experimental.pallas import tpu_sc as plsc

_B = 8
_M = 2048
_D = 1024
_TOTAL = 8192
_NW = 32
_MB = _B * _M // _NW
_C = 32
_NC = _MB // _C


def _make_sc_kernel():
  mesh = plsc.VectorSubcoreMesh(core_axis_name="c", subcore_axis_name="s")

  @functools.partial(
      pl.kernel,
      mesh=mesh,
      out_type=(
          jax.ShapeDtypeStruct((_B * _M, _D), jnp.float32),
          jax.ShapeDtypeStruct((_B * _M,), jnp.int32),
      ),
      scratch_types=[
          pltpu.VMEM((128,), jnp.int32),
          pltpu.VMEM((_NC, _C), jnp.int32),
          pltpu.VMEM((2, _C, _D), jnp.float32),
          pltpu.VMEM((_C, _D), jnp.float32),
          pltpu.VMEM((_MB,), jnp.int32),
          pltpu.SemaphoreType.DMA,
          pltpu.SemaphoreType.DMA,
      ],
  )
  def k(flat_hbm, cu_hbm, z_hbm, out_hbm, mask_hbm,
        cu_v, idx_v, bufs, zbuf, mbuf, gsem, wsem):
    wid = lax.axis_index("s") * 2 + lax.axis_index("c")
    pltpu.sync_copy(cu_hbm, cu_v)
    pltpu.sync_copy(z_hbm, zbuf)

    iota = lax.iota(jnp.int32, 16)
    cu_vec = cu_v[pl.ds(0, 16)]
    cus = [cu_vec[i] for i in range(_B + 1)]
    one = jnp.int32(1)
    zero = jnp.int32(0)

    b4 = wid // 4
    cu_b = jnp.where(b4 >= 1, cus[1] - cus[0], zero)
    lenb = jnp.where(b4 == 0, cus[1] - cus[0], zero)
    for t in range(1, _B):
      cu_b = cu_b + jnp.where(b4 >= t + 1, cus[t + 1] - cus[t], zero)
      lenb = lenb + jnp.where(b4 == t, cus[t + 1] - cus[t], zero)

    m0 = (wid % 4) * _MB
    obase = b4 * _M + m0
    lenv = jnp.zeros((16,), jnp.int32) + lenb

    for c in range(_NC):
      for h in range(2):
        mv = m0 + c * _C + h * 16 + iota
        idx_v[c, pl.ds(h * 16, 16)] = jnp.where(mv < lenv, cu_b + mv, zero)

    def not_full_pad(c):
      return lenb > m0 + c * _C

    def gather(c):
      pltpu.async_copy(flat_hbm.at[idx_v.at[c]], bufs.at[c % 2], gsem)

    def fire_write(c):
      lo = m0 + c * _C
      dst = out_hbm.at[pl.ds(obase + c * _C, _C)]

      @pl.when(not_full_pad(c))
      def _():
        pltpu.async_copy(bufs.at[c % 2], dst, wsem)

      @pl.when(jnp.logical_not(not_full_pad(c)))
      def _():
        pltpu.async_copy(zbuf, dst, wsem)

    def wait_write(c):
      pltpu.make_async_copy(
          zbuf, out_hbm.at[pl.ds(obase + c * _C, _C)], wsem).wait()

    @pl.when(not_full_pad(0))
    def _():
      gather(0)

    for c in range(_NC):
      lo = m0 + c * _C

      @pl.when(not_full_pad(c))
      def _():
        pltpu.make_async_copy(
            flat_hbm.at[idx_v.at[c]], bufs.at[c % 2], gsem).wait()

      straddle = jnp.logical_and(not_full_pad(c), lenb < lo + _C)

      @pl.when(straddle)
      def _():
        zv = jnp.zeros((16,), jnp.float32)

        def zrow(r, carry):
          for w in range(_D // 16):
            bufs[c % 2, r, pl.ds(w * 16, 16)] = zv
          return carry

        lax.fori_loop(lenb - lo, _C, zrow, zero)

      fire_write(c)
      if c + 1 < _NC:
        if c >= 1:
          wait_write(c - 1)

        @pl.when(not_full_pad(c + 1))
        def _():
          gather(c + 1)

    def mask_body(g, carry):
      mv = m0 + g * 16 + iota
      mbuf[pl.ds(g * 16, 16)] = jnp.where(mv >= lenv, one, zero)
      return carry

    lax.fori_loop(0, _MB // 16, mask_body, zero)
    pltpu.sync_copy(mbuf, mask_hbm.at[pl.ds(wid * _MB, _MB)])

    wait_write(_NC - 2)
    wait_write(_NC - 1)

  return k


_sc_pad = _make_sc_kernel()


@jax.jit
def kernel(flat, cu_seqlens):
  zeros = jnp.zeros((_C, _D), jnp.float32)
  cu16 = jnp.pad(cu_seqlens.astype(jnp.int32), (0, 128 - (_B + 1)))
  out_flat, mask_i = _sc_pad(flat, cu16, zeros)
  padded = out_flat.reshape(_B, _M, _D)
  mask = mask_i.reshape(_B, _M).astype(jnp.bool_)
  return padded, mask


# speedup vs baseline: 1.3370x; 1.3370x over previous
import functools

import jax
import jax.numpy as jnp
from jax import lax
from jax.experimental import pallas as pl
from jax.experimental.pallas import tpu as pltpu
from jax.experimental.pallas import tpu_sc as plsc

_B = 8
_M = 2048
_D = 1024
_TOTAL = 8192
_NW = 32
_VR = _TOTAL // _NW
_PR = (_B * _M - _TOTAL) // _NW
_C = 32
_NCV = _VR // _C
_CP = 16
_NCP = _PR // _CP
_NBUF = 3


def _make_sc_kernel():
  mesh = plsc.VectorSubcoreMesh(core_axis_name="c", subcore_axis_name="s")

  @functools.partial(
      pl.kernel,
      mesh=mesh,
      out_type=jax.ShapeDtypeStruct((_B * _M, _D), jnp.float32),
      scratch_types=[
          pltpu.VMEM((16,), jnp.int32),
          pltpu.VMEM((_NCV, _C), jnp.int32),
          pltpu.VMEM((_NCP, _CP), jnp.int32),
          pltpu.VMEM((_NBUF, _C, _D), jnp.float32),
          pltpu.VMEM((_CP, _D), jnp.float32),
          pltpu.SemaphoreType.DMA,
          pltpu.SemaphoreType.DMA,
          pltpu.SemaphoreType.DMA,
      ],
  )
  def k(flat_hbm, cu_hbm, out_hbm,
        cu_v, idxv, idxp, bufs, zbuf, gsem, ssem, psem):
    wid = lax.axis_index("s") * 2 + lax.axis_index("c")
    pltpu.sync_copy(cu_hbm, cu_v.at[pl.ds(0, _B + 1)])

    iota = lax.iota(jnp.int32, 16)
    cu_vec = cu_v[pl.ds(0, 16)]
    cus = [cu_vec[i] for i in range(_B + 1)]
    zero = jnp.int32(0)

    pbase = wid * _PR
    for c in range(_NCP):
      jv = pbase + c * _CP + iota
      cu_b1 = jnp.zeros((16,), jnp.int32) + cus[1]
      for t in range(1, _B):
        cu_b1 = cu_b1 + jnp.where(
            jv >= t * _M - cus[t], cus[t + 1] - cus[t], zero)
      idxp[c, pl.ds(0, _CP)] = cu_b1 + jv

    zv = jnp.zeros((16,), jnp.float32)

    def zrow(r, carry):
      for w in range(_D // 16):
        zbuf[r, pl.ds(w * 16, 16)] = zv
      return carry

    lax.fori_loop(0, _CP, zrow, zero)
    for c in range(_NCP):
      pltpu.async_copy(zbuf, out_hbm.at[idxp.at[c]], psem)

    vbase = wid * _VR
    for c in range(_NCV):
      for h in range(2):
        iv = vbase + c * _C + h * 16 + iota
        bm = jnp.where(iv >= cus[1], jnp.int32(_M), zero)
        cu_b = jnp.where(iv >= cus[1], cus[1] - cus[0], zero)
        for t in range(2, _B):
          bm = bm + jnp.where(iv >= cus[t], jnp.int32(_M), zero)
          cu_b = cu_b + jnp.where(iv >= cus[t], cus[t] - cus[t - 1], zero)
        idxv[c, pl.ds(h * 16, 16)] = bm + iv - cu_b

    def gather(c):
      pltpu.async_copy(
          flat_hbm.at[pl.ds(vbase + c * _C, _C)], bufs.at[c % _NBUF], gsem)

    def wait_gather(c):
      pltpu.make_async_copy(
          flat_hbm.at[pl.ds(vbase + c * _C, _C)], bufs.at[c % _NBUF],
          gsem).wait()

    def scatter(c):
      pltpu.async_copy(bufs.at[c % _NBUF], out_hbm.at[idxv.at[c]], ssem)

    def wait_scatter(c):
      pltpu.make_async_copy(
          bufs.at[c % _NBUF], out_hbm.at[idxv.at[c]], ssem).wait()

    for c in range(_NBUF):
      gather(c)
    for c in range(_NCV):
      wait_gather(c)
      scatter(c)
      if c + _NBUF < _NCV:
        wait_scatter(c)
        gather(c + _NBUF)

    for c in range(_NCV - _NBUF, _NCV):
      wait_scatter(c)
    for c in range(_NCP):
      pltpu.make_async_copy(zbuf, out_hbm.at[idxp.at[c]], psem).wait()

  return k


def _tc_mask_body(cu_ref, mask_ref):
  m = lax.broadcasted_iota(jnp.int32, (_B, _M), 1)
  row = lax.broadcasted_iota(jnp.int32, (_B, _M), 0)
  acc = jnp.zeros((_B, _M), jnp.bool_)
  for b in range(_B):
    lenb = cu_ref[b + 1] - cu_ref[b]
    acc = jnp.logical_or(acc, jnp.logical_and(row == b, m >= lenb))
  mask_ref[...] = acc


_sc_pad = _make_sc_kernel()

_tc_mask = pl.pallas_call(
    _tc_mask_body,
    out_shape=jax.ShapeDtypeStruct((_B, _M), jnp.bool_),
    in_specs=[pl.BlockSpec(memory_space=pltpu.SMEM)],
)


@jax.jit
def kernel(flat, cu_seqlens):
  cu = cu_seqlens.astype(jnp.int32)
  out_flat = _sc_pad(flat, cu)
  mask = _tc_mask(cu)
  return out_flat.reshape(_B, _M, _D), mask


# speedup vs baseline: 1.4510x; 1.0853x over previous
import functools

import jax
import jax.numpy as jnp
from jax import lax
from jax.experimental import pallas as pl
from jax.experimental.pallas import tpu as pltpu
from jax.experimental.pallas import tpu_sc as plsc

_B = 8
_M = 2048
_D = 1024
_TOTAL = 8192
_NPAD = _B * _M - _TOTAL
_NW = 32
_VR = _TOTAL // _NW
_C = 32
_NCV = _VR // _C
_NBUF = 3
_ZC = 256


def _make_sc_kernel():
  mesh = plsc.VectorSubcoreMesh(core_axis_name="c", subcore_axis_name="s")

  @functools.partial(
      pl.kernel,
      mesh=mesh,
      out_type=(),
      scratch_types=[
          pltpu.VMEM((16,), jnp.int32),
          pltpu.VMEM((_NCV, _C), jnp.int32),
          pltpu.VMEM((_NBUF, _C, _D), jnp.float32),
          pltpu.SemaphoreType.DMA,
          pltpu.SemaphoreType.DMA,
      ],
  )
  def k(flat_hbm, cu_hbm, out_hbm, cu_v, idxv, bufs, gsem, ssem):
    wid = lax.axis_index("s") * 2 + lax.axis_index("c")
    pltpu.sync_copy(cu_hbm, cu_v.at[pl.ds(0, _B + 1)])

    iota = lax.iota(jnp.int32, 16)
    cu_vec = cu_v[pl.ds(0, 16)]
    cus = [cu_vec[i] for i in range(_B + 1)]
    zero = jnp.int32(0)

    vbase = wid * _VR

    def valid_idx(g, carry):
      iv = vbase + g * 16 + iota
      bm = jnp.where(iv >= cus[1], jnp.int32(_M), zero)
      cu_b = jnp.where(iv >= cus[1], cus[1] - cus[0], zero)
      for t in range(2, _B):
        bm = bm + jnp.where(iv >= cus[t], jnp.int32(_M), zero)
        cu_b = cu_b + jnp.where(iv >= cus[t], cus[t] - cus[t - 1], zero)
      idxv[g // 2, pl.ds((g % 2) * 16, 16)] = bm + iv - cu_b
      return carry

    lax.fori_loop(0, 2 * _NCV, valid_idx, zero)

    def gather(c, slot):
      pltpu.async_copy(
          flat_hbm.at[pl.ds(vbase + c * _C, _C)], bufs.at[slot], gsem)

    def wait_gather(c, slot):
      pltpu.make_async_copy(
          flat_hbm.at[pl.ds(vbase + c * _C, _C)], bufs.at[slot], gsem).wait()

    def scatter(c, slot):
      pltpu.async_copy(bufs.at[slot], out_hbm.at[idxv.at[c]], ssem)

    def wait_scatter(c, slot):
      pltpu.make_async_copy(
          bufs.at[slot], out_hbm.at[idxv.at[c]], ssem).wait()

    for c in range(_NBUF):
      gather(c, c)

    def pipe(c, slot):
      wait_gather(c, slot)
      scatter(c, slot)

      @pl.when(c + _NBUF < _NCV)
      def _():
        wait_scatter(c, slot)
        gather(c + _NBUF, slot)

      return jnp.where(slot == _NBUF - 1, 0, slot + 1)

    lax.fori_loop(0, _NCV, pipe, zero)

    def drain(c, carry):
      wait_scatter(c, jnp.mod(c, _NBUF))
      return carry

    lax.fori_loop(_NCV - _NBUF, _NCV, drain, zero)

  return k


def _tc_zero_body(cu_ref, out_ref, zc, sem):
  zc[...] = jnp.zeros((_ZC, _D), jnp.float32)
  for fire in (True, False):
    for b in range(_B):
      lenb = cu_ref[b + 1] - cu_ref[b]
      start = (lenb // 8) * 8
      length = _M - start
      big = length // _ZC
      end = (b + 1) * _M

      def zchunk(i, carry):
        cp = pltpu.make_async_copy(
            zc,
            out_ref.at[pl.ds(pl.multiple_of(end - (i + 1) * _ZC, 8), _ZC)],
            sem)
        if fire:
          cp.start()
        else:
          cp.wait()
        return carry

      lax.fori_loop(0, big, zchunk, jnp.int32(0))

      pos = end - big * _ZC
      rem = length - big * _ZC
      kk = _ZC // 2
      while kk >= 8:
        k = kk
        hit = (rem & k) != 0
        pos = jnp.where(hit, pos - k, pos)

        @pl.when(hit)
        def _(pos=pos, k=k):
          cp = pltpu.make_async_copy(
              zc.at[pl.ds(0, k)],
              out_ref.at[pl.ds(pl.multiple_of(pos, 8), k)], sem)
          if fire:
            cp.start()
          else:
            cp.wait()

        kk //= 2


_sc_valid = _make_sc_kernel()

_tc_zero = pl.pallas_call(
    _tc_zero_body,
    out_shape=jax.ShapeDtypeStruct((_B * _M, _D), jnp.float32),
    in_specs=[pl.BlockSpec(memory_space=pltpu.SMEM)],
    out_specs=pl.BlockSpec(memory_space=pl.ANY),
    scratch_shapes=[
        pltpu.VMEM((_ZC, _D), jnp.float32),
        pltpu.SemaphoreType.DMA,
    ],
)


def _tc_mask_body(cu_ref, mask_ref):
  m = lax.broadcasted_iota(jnp.int32, (_B, _M), 1)
  row = lax.broadcasted_iota(jnp.int32, (_B, _M), 0)
  acc = jnp.zeros((_B, _M), jnp.bool_)
  for b in range(_B):
    lenb = cu_ref[b + 1] - cu_ref[b]
    acc = jnp.logical_or(acc, jnp.logical_and(row == b, m >= lenb))
  mask_ref[...] = acc


_tc_mask = pl.pallas_call(
    _tc_mask_body,
    out_shape=jax.ShapeDtypeStruct((_B, _M), jnp.bool_),
    in_specs=[pl.BlockSpec(memory_space=pltpu.SMEM)],
)


@jax.jit
def kernel(flat, cu_seqlens):
  cu = cu_seqlens.astype(jnp.int32)
  init = _tc_zero(cu)
  ref = jax.new_ref(init)
  _sc_valid(flat, cu, ref)
  mask = _tc_mask(cu)
  return ref[...].reshape(_B, _M, _D), mask
